# baseline (speedup 1.0000x reference)
import functools

import jax
import jax.numpy as jnp
from jax import lax
from jax.experimental import pallas as pl
from jax.experimental.pallas import tpu as pltpu
from jax.experimental.pallas import tpu_sc as plsc

N_ROWS = 16384
SC_ROWS = N_ROWS // 2
N_COLS = 1000
LANES = 16
NUM_FULL = N_COLS // LANES
TAIL_OFF = N_COLS - LANES
TAIL_DUP = LANES - (N_COLS - NUM_FULL * LANES)
NC, NS = 2, 16
NW = NC * NS
ROWS_PER_W = SC_ROWS // NW
CHUNK = 32
NCHUNK = ROWS_PER_W // CHUNK
SPLIT = NUM_FULL // 2
NEG_INF = float("-inf")

_GATHER_DNUMS = lax.GatherDimensionNumbers(
    offset_dims=(), collapsed_slice_dims=(0,), start_index_map=(0,))


def _shuf(v, lane, sh):
  return lax.gather(v, (lane ^ sh)[:, None], _GATHER_DNUMS, (1,),
                    mode=lax.GatherScatterMode.PROMISE_IN_BOUNDS)


def _allmax(v, lane):
  for sh in (1, 2, 4, 8):
    v = jnp.maximum(v, _shuf(v, lane, sh))
  return v


def _allmin(v, lane):
  for sh in (1, 2, 4, 8):
    v = jnp.minimum(v, _shuf(v, lane, sh))
  return v


def _sc_weak_target(weak):
  mesh = plsc.VectorSubcoreMesh(core_axis_name="c", subcore_axis_name="s")

  @functools.partial(
      pl.kernel,
      mesh=mesh,
      compiler_params=pltpu.CompilerParams(
          use_tc_tiling_on_sc=True, needs_layout_passes=False),
      out_type=jax.ShapeDtypeStruct((SC_ROWS,), jnp.int32),
      scratch_types=[
          pltpu.VMEM((CHUNK, N_COLS), jnp.float32),
          pltpu.VMEM((CHUNK, N_COLS), jnp.float32),
          pltpu.VMEM((ROWS_PER_W,), jnp.int32),
          pltpu.SemaphoreType.DMA,
          pltpu.SemaphoreType.DMA,
      ],
  )
  def body(weak_hbm, tgt_hbm, wbufA, wbufB, t_all, semA, semB):
    wid = lax.axis_index("s") * NC + lax.axis_index("c")
    lane = lax.iota(jnp.int32, LANES)
    row_base = wid * ROWS_PER_W

    def src(ch):
      row0 = row_base + ch * CHUNK
      return weak_hbm.at[pl.ds(row0, CHUNK), :]

    def compute_chunk(ch, wbuf):
      def half(h):
        def row_body(r0, ivec):
          r = h * LANES + r0
          mwA = jnp.full((LANES,), NEG_INF, jnp.float32)
          mwB = mwA
          jwA = jnp.zeros((LANES,), jnp.int32)
          jwB = jwA
          for j in range(NUM_FULL):
            wv = wbuf[r, pl.ds(j * LANES, LANES)]
            if j < SPLIT:
              take = wv > mwA
              mwA = jnp.maximum(mwA, wv)
              jwA = jnp.where(take, j, jwA)
            else:
              take = wv > mwB
              mwB = jnp.maximum(mwB, wv)
              jwB = jnp.where(take, j, jwB)
          takeB = mwB > mwA
          m_w = jnp.maximum(mwA, mwB)
          j_w = jnp.where(takeB, jwB, jwA)
          wv = wbuf[r, pl.ds(TAIL_OFF, LANES)]
          wv = jnp.where(lane >= TAIL_DUP, wv, NEG_INF)
          take = wv > m_w
          m_w = jnp.maximum(m_w, wv)
          j_w = jnp.where(take, NUM_FULL, j_w)
          col = j_w * LANES + lane
          col = jnp.where(j_w == NUM_FULL, col - TAIL_DUP, col)
          mw_max = _allmax(m_w, lane)
          cand = jnp.where(m_w == mw_max, col, jnp.int32(N_COLS))
          target = _allmin(cand, lane)
          return jnp.where(lane == r0, target, ivec)

        ivec = lax.fori_loop(0, LANES, row_body,
                             jnp.zeros((LANES,), jnp.int32))
        t_all[pl.ds(ch * CHUNK + h * LANES, LANES)] = ivec

      half(0)
      half(1)

    pltpu.async_copy(src(0), wbufA, semA)

    def pair_body(i, carry):
      ch = 2 * i
      pltpu.async_copy(src(ch + 1), wbufB, semB)
      pltpu.make_async_copy(src(ch), wbufA, semA).wait()
      compute_chunk(ch, wbufA)

      @pl.when(ch + 2 < NCHUNK)
      def _():
        pltpu.async_copy(src(ch + 2), wbufA, semA)

      pltpu.make_async_copy(src(ch + 1), wbufB, semB).wait()
      compute_chunk(ch + 1, wbufB)
      return carry

    lax.fori_loop(0, NCHUNK // 2, pair_body, 0)

    pltpu.sync_copy(t_all, tgt_hbm.at[pl.ds(row_base, ROWS_PER_W)])

  return body(weak)


_TC_BLOCK = 1024


def _tc_strong(strong, tgt):
  def body(s_ref, t_ref, out_ref):
    @pl.when(pl.program_id(0) == 0)
    def _():
      out_ref[0, 0] = jnp.float32(0.0)

    s = s_ref[...]
    t = t_ref[...]
    m = jnp.max(s, axis=1, keepdims=True)
    lse = m[:, 0] + jnp.log(jnp.sum(jnp.exp(s - m), axis=1))
    cols = lax.broadcasted_iota(jnp.int32, (_TC_BLOCK, N_COLS), 1)
    tv = jnp.sum(jnp.where(cols == t, s, 0.0))
    out_ref[0, 0] += jnp.sum(lse) - tv

  return pl.pallas_call(
      body,
      grid=(N_ROWS // _TC_BLOCK,),
      in_specs=[pl.BlockSpec((_TC_BLOCK, N_COLS), lambda i: (i, 0)),
                pl.BlockSpec((_TC_BLOCK, 1), lambda i: (i, 0))],
      out_specs=pl.BlockSpec((1, 1), lambda i: (0, 0),
                             memory_space=pltpu.SMEM),
      out_shape=jax.ShapeDtypeStruct((1, 1), jnp.float32),
  )(strong, tgt)


_AM_BLOCK = 1024


def _tc_argmax_upper(weak):
  nblk = (N_ROWS - SC_ROWS) // _AM_BLOCK

  def body(w_ref, out_ref):
    w = w_ref[...]
    m = jnp.max(w, axis=1, keepdims=True)
    cols = lax.broadcasted_iota(jnp.int32, (_AM_BLOCK, N_COLS), 1)
    out_ref[...] = jnp.min(jnp.where(w == m, cols, jnp.int32(N_COLS)),
                           axis=1, keepdims=True)

  return pl.pallas_call(
      body,
      grid=(nblk,),
      in_specs=[pl.BlockSpec((_AM_BLOCK, N_COLS),
                             lambda i: (i + SC_ROWS // _AM_BLOCK, 0))],
      out_specs=pl.BlockSpec((_AM_BLOCK, 1), lambda i: (i, 0)),
      out_shape=jax.ShapeDtypeStruct((N_ROWS - SC_ROWS, 1), jnp.int32),
  )(weak)


@jax.jit
def _impl(anchors_weak, anchors_strong):
  tgt_lo = _sc_weak_target(anchors_weak)
  tgt_hi = _tc_argmax_upper(anchors_weak)
  tgt = jnp.concatenate([tgt_lo.reshape(SC_ROWS, 1), tgt_hi], axis=0)
  total = _tc_strong(anchors_strong, tgt)
  return total[0, 0] * (1.0 / N_ROWS)


def kernel(head_id, anchors_weak, anchors_strong):
  del head_id
  return _impl(anchors_weak, anchors_strong)


# speedup vs baseline: 1.1031x; 1.1031x over previous
import functools

import jax
import jax.numpy as jnp
from jax import lax
from jax.experimental import pallas as pl
from jax.experimental.pallas import tpu as pltpu
from jax.experimental.pallas import tpu_sc as plsc

N_ROWS = 16384
N_COLS = 1000
LANES = 16
NUM_FULL = N_COLS // LANES
TAIL_OFF = N_COLS - LANES
TAIL_DUP = LANES - (N_COLS - NUM_FULL * LANES)
NC, NS = 2, 16
NW = NC * NS
ROWS_PER_W = N_ROWS // NW
CHUNK = 16
NCHUNK = ROWS_PER_W // CHUNK
SPLIT = NUM_FULL // 2
NEG_INF = float("-inf")

_GATHER_DNUMS = lax.GatherDimensionNumbers(
    offset_dims=(), collapsed_slice_dims=(0,), start_index_map=(0,))


def _shuf(v, lane, sh):
  return lax.gather(v, (lane ^ sh)[:, None], _GATHER_DNUMS, (1,),
                    mode=lax.GatherScatterMode.PROMISE_IN_BOUNDS)


def _allmax(v, lane):
  for sh in (1, 2, 4, 8):
    v = jnp.maximum(v, _shuf(v, lane, sh))
  return v


def _allmin(v, lane):
  for sh in (1, 2, 4, 8):
    v = jnp.minimum(v, _shuf(v, lane, sh))
  return v


def _sc_weak_target(weak):
  mesh = plsc.VectorSubcoreMesh(core_axis_name="c", subcore_axis_name="s")

  @functools.partial(
      pl.kernel,
      mesh=mesh,
      compiler_params=pltpu.CompilerParams(
          use_tc_tiling_on_sc=True, needs_layout_passes=False),
      out_type=jax.ShapeDtypeStruct((N_ROWS,), jnp.int32),
      scratch_types=[
          pltpu.VMEM((CHUNK, N_COLS), jnp.float32),
          pltpu.VMEM((CHUNK, N_COLS), jnp.float32),
          pltpu.VMEM((ROWS_PER_W,), jnp.int32),
          pltpu.SemaphoreType.DMA,
          pltpu.SemaphoreType.DMA,
      ],
  )
  def body(weak_hbm, tgt_hbm, wbufA, wbufB, t_all, semA, semB):
    wid = lax.axis_index("s") * NC + lax.axis_index("c")
    lane = lax.iota(jnp.int32, LANES)
    row_base = wid * ROWS_PER_W

    def src(ch):
      row0 = row_base + ch * CHUNK
      return weak_hbm.at[pl.ds(row0, CHUNK), :]

    def compute_chunk(ch, wbuf):
      def half(h):
        def row_body(r0, ivec):
          r = h * LANES + r0
          mwA = jnp.full((LANES,), NEG_INF, jnp.float32)
          mwB = mwA
          jwA = jnp.zeros((LANES,), jnp.int32)
          jwB = jwA
          for j in range(NUM_FULL):
            wv = wbuf[r, pl.ds(j * LANES, LANES)]
            if j < SPLIT:
              take = wv > mwA
              mwA = jnp.maximum(mwA, wv)
              jwA = jnp.where(take, j, jwA)
            else:
              take = wv > mwB
              mwB = jnp.maximum(mwB, wv)
              jwB = jnp.where(take, j, jwB)
          takeB = mwB > mwA
          m_w = jnp.maximum(mwA, mwB)
          j_w = jnp.where(takeB, jwB, jwA)
          wv = wbuf[r, pl.ds(TAIL_OFF, LANES)]
          wv = jnp.where(lane >= TAIL_DUP, wv, NEG_INF)
          take = wv > m_w
          m_w = jnp.maximum(m_w, wv)
          j_w = jnp.where(take, NUM_FULL, j_w)
          col = j_w * LANES + lane
          col = jnp.where(j_w == NUM_FULL, col - TAIL_DUP, col)
          mw_max = _allmax(m_w, lane)
          cand = jnp.where(m_w == mw_max, col, jnp.int32(N_COLS))
          target = _allmin(cand, lane)
          return jnp.where(lane == r0, target, ivec)

        ivec = lax.fori_loop(0, LANES, row_body,
                             jnp.zeros((LANES,), jnp.int32))
        t_all[pl.ds(ch * CHUNK + h * LANES, LANES)] = ivec

      half(0)

    pltpu.async_copy(src(0), wbufA, semA)

    def pair_body(i, carry):
      ch = 2 * i
      pltpu.async_copy(src(ch + 1), wbufB, semB)
      pltpu.make_async_copy(src(ch), wbufA, semA).wait()
      compute_chunk(ch, wbufA)

      @pl.when(ch + 2 < NCHUNK)
      def _():
        pltpu.async_copy(src(ch + 2), wbufA, semA)

      pltpu.make_async_copy(src(ch + 1), wbufB, semB).wait()
      compute_chunk(ch + 1, wbufB)
      return carry

    lax.fori_loop(0, NCHUNK // 2, pair_body, 0)

    pltpu.sync_copy(t_all, tgt_hbm.at[pl.ds(row_base, ROWS_PER_W)])

  return body(weak)


_TC_BLOCK = 1024


def _tc_strong(strong, tgt):
  def body(s_ref, t_ref, out_ref):
    @pl.when(pl.program_id(0) == 0)
    def _():
      out_ref[0, 0] = jnp.float32(0.0)

    s = s_ref[...]
    t = t_ref[...]
    m = jnp.max(s, axis=1, keepdims=True)
    lse = m[:, 0] + jnp.log(jnp.sum(jnp.exp(s - m), axis=1))
    cols = lax.broadcasted_iota(jnp.int32, (_TC_BLOCK, N_COLS), 1)
    tv = jnp.sum(jnp.where(cols == t, s, 0.0))
    out_ref[0, 0] += jnp.sum(lse) - tv

  return pl.pallas_call(
      body,
      grid=(N_ROWS // _TC_BLOCK,),
      in_specs=[pl.BlockSpec((_TC_BLOCK, N_COLS), lambda i: (i, 0)),
                pl.BlockSpec((_TC_BLOCK, 1), lambda i: (i, 0))],
      out_specs=pl.BlockSpec((1, 1), lambda i: (0, 0),
                             memory_space=pltpu.SMEM),
      out_shape=jax.ShapeDtypeStruct((1, 1), jnp.float32),
  )(strong, tgt)


@jax.jit
def _impl(anchors_weak, anchors_strong):
  tgt = _sc_weak_target(anchors_weak)
  total = _tc_strong(anchors_strong, tgt.reshape(N_ROWS, 1))
  return total[0, 0] * (1.0 / N_ROWS)


def kernel(head_id, anchors_weak, anchors_strong):
  del head_id
  return _impl(anchors_weak, anchors_strong)


# speedup vs baseline: 1.1102x; 1.0064x over previous
import functools

import jax
import jax.numpy as jnp
from jax import lax
from jax.experimental import pallas as pl
from jax.experimental.pallas import tpu as pltpu
from jax.experimental.pallas import tpu_sc as plsc

N_ROWS = 16384
N_COLS = 1000
LANES = 16
NUM_FULL = N_COLS // LANES
TAIL_OFF = N_COLS - LANES
TAIL_DUP = LANES - (N_COLS - NUM_FULL * LANES)
NC, NS = 2, 16
NW = NC * NS
ROWS_PER_W = N_ROWS // NW
CHUNK = 16
NCHUNK = ROWS_PER_W // CHUNK
SPLIT = NUM_FULL // 2
NEG_INF = float("-inf")

_GATHER_DNUMS = lax.GatherDimensionNumbers(
    offset_dims=(), collapsed_slice_dims=(0,), start_index_map=(0,))


def _shuf(v, lane, sh):
  return lax.gather(v, (lane ^ sh)[:, None], _GATHER_DNUMS, (1,),
                    mode=lax.GatherScatterMode.PROMISE_IN_BOUNDS)


def _allmax(v, lane):
  for sh in (1, 2, 4, 8):
    v = jnp.maximum(v, _shuf(v, lane, sh))
  return v


def _allmin(v, lane):
  for sh in (1, 2, 4, 8):
    v = jnp.minimum(v, _shuf(v, lane, sh))
  return v


def _sc_weak_target(weak):
  mesh = plsc.VectorSubcoreMesh(core_axis_name="c", subcore_axis_name="s")

  @functools.partial(
      pl.kernel,
      mesh=mesh,
      compiler_params=pltpu.CompilerParams(
          use_tc_tiling_on_sc=True, needs_layout_passes=False),
      out_type=jax.ShapeDtypeStruct((N_ROWS,), jnp.int32),
      scratch_types=[
          pltpu.VMEM((CHUNK, N_COLS), jnp.float32),
          pltpu.VMEM((CHUNK, N_COLS), jnp.float32),
          pltpu.VMEM((ROWS_PER_W,), jnp.int32),
          pltpu.SemaphoreType.DMA,
          pltpu.SemaphoreType.DMA,
      ],
  )
  def body(weak_hbm, tgt_hbm, wbufA, wbufB, t_all, semA, semB):
    wid = lax.axis_index("s") * NC + lax.axis_index("c")
    lane = lax.iota(jnp.int32, LANES)
    row_base = wid * ROWS_PER_W

    def src(ch):
      row0 = row_base + ch * CHUNK
      return weak_hbm.at[pl.ds(row0, CHUNK), :]

    def compute_chunk(ch, wbuf):
      def half(h):
        def row_body(r0, ivec):
          r = h * LANES + r0
          mwA = jnp.full((LANES,), NEG_INF, jnp.float32)
          mwB = mwA
          jwA = jnp.zeros((LANES,), jnp.int32)
          jwB = jwA
          for j in range(NUM_FULL):
            wv = wbuf[r, pl.ds(j * LANES, LANES)]
            if j < SPLIT:
              take = wv > mwA
              mwA = jnp.maximum(mwA, wv)
              jwA = jnp.where(take, j, jwA)
            else:
              take = wv > mwB
              mwB = jnp.maximum(mwB, wv)
              jwB = jnp.where(take, j, jwB)
          takeB = mwB > mwA
          m_w = jnp.maximum(mwA, mwB)
          j_w = jnp.where(takeB, jwB, jwA)
          wv = wbuf[r, pl.ds(TAIL_OFF, LANES)]
          wv = jnp.where(lane >= TAIL_DUP, wv, NEG_INF)
          take = wv > m_w
          m_w = jnp.maximum(m_w, wv)
          j_w = jnp.where(take, NUM_FULL, j_w)
          col = j_w * LANES + lane
          col = jnp.where(j_w == NUM_FULL, col - TAIL_DUP, col)
          mw_max = _allmax(m_w, lane)
          cand = jnp.where(m_w == mw_max, col, jnp.int32(N_COLS))
          target = _allmin(cand, lane)
          return jnp.where(lane == r0, target, ivec)

        ivec = lax.fori_loop(0, LANES, row_body,
                             jnp.zeros((LANES,), jnp.int32))
        t_all[pl.ds(ch * CHUNK + h * LANES, LANES)] = ivec

      half(0)

    pltpu.async_copy(src(0), wbufA, semA)

    def pair_body(i, carry):
      ch = 2 * i
      pltpu.async_copy(src(ch + 1), wbufB, semB)
      pltpu.make_async_copy(src(ch), wbufA, semA).wait()
      compute_chunk(ch, wbufA)

      @pl.when(ch + 2 < NCHUNK)
      def _():
        pltpu.async_copy(src(ch + 2), wbufA, semA)

      pltpu.make_async_copy(src(ch + 1), wbufB, semB).wait()
      compute_chunk(ch + 1, wbufB)
      return carry

    lax.fori_loop(0, NCHUNK // 2, pair_body, 0)

    pltpu.sync_copy(t_all, tgt_hbm.at[pl.ds(row_base, ROWS_PER_W)])

  return body(weak)


_TC_BLOCK = 1024


def _tc_strong(strong, tgt):
  def body(s_ref, t_ref, out_ref):
    @pl.when(pl.program_id(0) == 0)
    def _():
      out_ref[0, 0] = jnp.float32(0.0)

    s = s_ref[...]
    t = t_ref[...]
    m = jnp.max(s, axis=1, keepdims=True)
    lse = m[:, 0] + jnp.log(jnp.sum(jnp.exp(s - m), axis=1))
    cols = lax.broadcasted_iota(jnp.int32, (_TC_BLOCK, N_COLS), 1)
    tv = jnp.sum(jnp.where(cols == t, s, 0.0))
    out_ref[0, 0] += (jnp.sum(lse) - tv) * (1.0 / N_ROWS)

  return pl.pallas_call(
      body,
      grid=(N_ROWS // _TC_BLOCK,),
      in_specs=[pl.BlockSpec((_TC_BLOCK, N_COLS), lambda i: (i, 0)),
                pl.BlockSpec((_TC_BLOCK, 1), lambda i: (i, 0))],
      out_specs=pl.BlockSpec((1, 1), lambda i: (0, 0),
                             memory_space=pltpu.SMEM),
      out_shape=jax.ShapeDtypeStruct((1, 1), jnp.float32),
  )(strong, tgt)


@jax.jit
def _impl(anchors_weak, anchors_strong):
  tgt = _sc_weak_target(anchors_weak)
  total = _tc_strong(anchors_strong, tgt.reshape(N_ROWS, 1))
  return total[0, 0]


def kernel(head_id, anchors_weak, anchors_strong):
  del head_id
  return _impl(anchors_weak, anchors_strong)
